# Initial kernel scaffold; baseline (speedup 1.0000x reference)
import jax
import jax.numpy as jnp
from jax.experimental import pallas as pl


def kernel(boxes, scores, bases):
    raise NotImplementedError("write your pallas kernel here")


# speedup vs baseline: 14.5509x; 14.5509x over previous
import functools

import jax
import jax.numpy as jnp
from jax import lax
from jax.experimental import pallas as pl
from jax.experimental.pallas import tpu as pltpu

N = 20000
IMG_W = 1333.0
IMG_H = 800.0
SCORE_THRESH = 0.5
NMS_THRESH = 0.5
PRE_NMS_TOPK = 2000
TOPK_PER_IMAGE = 100

NC = 2048
BLK = 128
NBLK = NC // BLK


def _nms_kernel(x1c_ref, y1c_ref, x2c_ref, y2c_ref,
                x1r_ref, y1r_ref, x2r_ref, y2r_ref,
                valid_ref, keep_out_ref, keep_ref, t_ref):
    x1r = x1r_ref[:, :]
    y1r = y1r_ref[:, :]
    x2r = x2r_ref[:, :]
    y2r = y2r_ref[:, :]
    area_r = jnp.clip(x2r - x1r, 0.0) * jnp.clip(y2r - y1r, 0.0)

    keep_ref[:, :] = valid_ref[:, :]

    col = lax.broadcasted_iota(jnp.int32, (1, NC), 1)
    lane = lax.broadcasted_iota(jnp.int32, (1, BLK), 1)

    for bi in range(NBLK):
        base = bi * BLK
        x1c = x1c_ref[pl.ds(base, BLK), :]
        y1c = y1c_ref[pl.ds(base, BLK), :]
        x2c = x2c_ref[pl.ds(base, BLK), :]
        y2c = y2c_ref[pl.ds(base, BLK), :]
        area_c = jnp.clip(x2c - x1c, 0.0) * jnp.clip(y2c - y1c, 0.0)

        ltx = jnp.maximum(x1c, x1r)
        lty = jnp.maximum(y1c, y1r)
        rbx = jnp.minimum(x2c, x2r)
        rby = jnp.minimum(y2c, y2r)
        wx = jnp.clip(rbx - ltx, 0.0)
        wy = jnp.clip(rby - lty, 0.0)
        inter = wx * wy
        iou = inter / (area_c + area_r - inter + 1e-9)
        over = (iou > NMS_THRESH).astype(jnp.float32)

        t_ref[:, :] = over[:, base:base + BLK]
        keep_blk = keep_ref[:, base:base + BLK]

        def body(i, kb):
            cur = jnp.max(jnp.where(lane == i, kb, 0.0))
            row = t_ref[pl.ds(i, 1), :]
            sup = row * cur * (lane > i).astype(jnp.float32)
            return kb * (1.0 - sup)

        keep_blk = lax.fori_loop(0, BLK, body, keep_blk)
        keep_ref[:, base:base + BLK] = keep_blk

        if bi + 1 < NBLK:
            kcol = keep_blk.reshape(BLK, 1)
            sup_all = jnp.max(over * kcol, axis=0, keepdims=True)
            sup_all = jnp.where(col >= base + BLK, sup_all, 0.0)
            keep_ref[:, :] = keep_ref[:, :] * (1.0 - sup_all)

    keep_out_ref[:, :] = keep_ref[:, :]


def _run_nms(b, valid_f):
    cols = [b[:, k:k + 1] for k in range(4)]
    rows = [b[:, k].reshape(1, NC) for k in range(4)]
    keep = pl.pallas_call(
        _nms_kernel,
        out_shape=jax.ShapeDtypeStruct((1, NC), jnp.float32),
        scratch_shapes=[
            pltpu.VMEM((1, NC), jnp.float32),
            pltpu.VMEM((BLK, BLK), jnp.float32),
        ],
    )(*cols, *rows, valid_f.reshape(1, NC))
    return keep[0]


def _delta_to_bases(b6, boxes):
    x1 = boxes[:, 0]; y1 = boxes[:, 1]; x2 = boxes[:, 2]; y2 = boxes[:, 3]
    dx = x2 - x1
    dy = y2 - y1
    midx = (x1 + x2) / 2.0 + b6[:, 0] * dx
    midy = (y1 + y2) / 2.0 + b6[:, 1] * dy
    firstx = b6[:, 2]; firsty = b6[:, 3]; secondx = b6[:, 4]; secondy = b6[:, 5]
    X1 = midx + firstx * dx
    Y1 = midy + firsty * dy
    X2 = midx + secondx * dx
    Y2 = midy + secondy * dy
    X3 = midx - secondx * dx
    Y3 = midy - secondy * dy
    X4 = midx - firstx * dx
    Y4 = midy - firsty * dy
    return jnp.stack((X1, Y1, X2, Y2, X3, Y3, X4, Y4, midx, midy), axis=-1)


def kernel(boxes, scores, bases):
    s = scores[:, 0]
    bx = jnp.stack([
        jnp.clip(boxes[:, 0], 0.0, IMG_W),
        jnp.clip(boxes[:, 1], 0.0, IMG_H),
        jnp.clip(boxes[:, 2], 0.0, IMG_W),
        jnp.clip(boxes[:, 3], 0.0, IMG_H),
    ], axis=1)
    sm = jnp.where(s > SCORE_THRESH, s, -1.0)
    sc, idx = lax.top_k(sm, PRE_NMS_TOPK)
    b = bx[idx]
    bs = bases[idx]
    valid = sc > SCORE_THRESH

    pad = NC - PRE_NMS_TOPK
    b_p = jnp.concatenate([b, jnp.zeros((pad, 4), jnp.float32)], axis=0)
    valid_p = jnp.concatenate([valid, jnp.zeros((pad,), bool)], axis=0)

    keep_f = _run_nms(b_p, valid_p.astype(jnp.float32))[:PRE_NMS_TOPK]
    keep = keep_f > 0.5

    fs = jnp.where(keep, sc, -1.0)
    out_s, fidx = lax.top_k(fs, TOPK_PER_IMAGE)
    out_b = b[fidx]
    out_bs = bs[fidx]
    pred_h = out_bs[:, 6] * (out_b[:, 3] - out_b[:, 1]) + (out_b[:, 3] - out_b[:, 1])
    pred_bases = _delta_to_bases(out_bs[:, :6], out_b)
    return jnp.concatenate(
        [pred_bases, pred_h[:, None], out_b, out_s[:, None]], axis=1)
